# baseline (speedup 1.0000x reference)
import functools

import jax
import jax.numpy as jnp
from jax import lax
from jax.experimental import pallas as pl
from jax.experimental.pallas import tpu as pltpu
from jax.experimental.pallas import tpu_sc as plsc

BATCH = 16384
EMBED = 64
NC = 2
NS = 16
NW = NC * NS
B_PER_W = BATCH // NW
CHUNK = 128
N_CHUNK = B_PER_W // CHUNK
GROUPS = B_PER_W // 16


def _body(uid_hbm, pid_hbm, utab_hbm, ptab_hbm, wb_hbm, out_hbm,
          uidx_v, pidx_v, urows_v, prows_v, wb_v, out_v, sem):
    wid = lax.axis_index("s") * NC + lax.axis_index("c")
    base = wid * B_PER_W

    for j in range(N_CHUNK):
        pltpu.sync_copy(uid_hbm.at[pl.ds(base + j * CHUNK, CHUNK)],
                        uidx_v.at[j])
        pltpu.sync_copy(pid_hbm.at[pl.ds(base + j * CHUNK, CHUNK)],
                        pidx_v.at[j])
    pltpu.sync_copy(wb_hbm, wb_v)

    copies = []
    for j in range(N_CHUNK):
        copies.append(pltpu.async_copy(
            utab_hbm.at[uidx_v.at[j]],
            urows_v.at[pl.ds(j * CHUNK, CHUNK)], sem))
        copies.append(pltpu.async_copy(
            ptab_hbm.at[pidx_v.at[j]],
            prows_v.at[pl.ds(j * CHUNK, CHUNK)], sem))
    for c in copies:
        c.wait()

    wc = [wb_v[pl.ds(c * 16, 16)] for c in range(EMBED // 16)]
    bias_vec = wb_v[pl.ds(EMBED, 16)]
    lane = lax.iota(jnp.int32, 16)

    def group(g, carry):
        rows = g * 16 + lane
        acc = jnp.zeros((16,), jnp.float32)
        for d in range(EMBED):
            dsplat = jnp.full((16,), d, jnp.int32)
            uv = plsc.load_gather(urows_v, [rows, dsplat])
            pv = plsc.load_gather(prows_v, [rows, dsplat])
            acc = acc + uv * pv * wc[d // 16][d % 16]
        out_v[pl.ds(g * 16, 16)] = acc + bias_vec
        return carry

    lax.fori_loop(0, GROUPS, group, 0)
    pltpu.sync_copy(out_v, out_hbm.at[pl.ds(base, B_PER_W)])


@jax.jit
def _run(uid, pid, utab, ptab, wb):
    mesh = plsc.VectorSubcoreMesh(core_axis_name="c", subcore_axis_name="s",
                                  num_cores=NC, num_subcores=NS)
    f = pl.kernel(
        _body,
        out_type=jax.ShapeDtypeStruct((BATCH,), jnp.float32),
        mesh=mesh,
        compiler_params=pltpu.CompilerParams(needs_layout_passes=False,
                                             use_tc_tiling_on_sc=False),
        scratch_types=[
            pltpu.VMEM((N_CHUNK, CHUNK), jnp.int32),
            pltpu.VMEM((N_CHUNK, CHUNK), jnp.int32),
            pltpu.VMEM((B_PER_W, EMBED), jnp.float32),
            pltpu.VMEM((B_PER_W, EMBED), jnp.float32),
            pltpu.VMEM((EMBED + 16,), jnp.float32),
            pltpu.VMEM((B_PER_W,), jnp.float32),
            pltpu.SemaphoreType.DMA,
        ],
    )
    return f(uid, pid, utab, ptab, wb)


def kernel(user_ids, product_ids, user_embedding, product_embedding, fc_w, fc_b):
    uid = user_ids.astype(jnp.int32)
    pid = product_ids.astype(jnp.int32)
    wb = jnp.concatenate(
        [fc_w.reshape(EMBED), jnp.broadcast_to(fc_b.reshape(1), (16,))])
    return _run(uid, pid, user_embedding, product_embedding, wb)


# speedup vs baseline: 1.0134x; 1.0134x over previous
import functools

import jax
import jax.numpy as jnp
from jax import lax
from jax.experimental import pallas as pl
from jax.experimental.pallas import tpu as pltpu
from jax.experimental.pallas import tpu_sc as plsc

BATCH = 16384
EMBED = 64
NC = 2
NS = 16
NW = NC * NS
B_PER_W = BATCH // NW
CHUNK = 128
N_CHUNK = B_PER_W // CHUNK

ROWS_BLK = 1024
N_BLK = BATCH // ROWS_BLK


def _gather_body(ids_hbm, tab_hbm, out_hbm, idx_v, rows_v, sem):
    wid = lax.axis_index("s") * NC + lax.axis_index("c")
    base = wid * B_PER_W
    for j in range(N_CHUNK):
        pltpu.sync_copy(ids_hbm.at[pl.ds(base + j * CHUNK, CHUNK)],
                        idx_v.at[j])
    copies = [
        pltpu.async_copy(tab_hbm.at[idx_v.at[j]],
                         rows_v.at[pl.ds(j * CHUNK, CHUNK)], sem)
        for j in range(N_CHUNK)
    ]
    for c in copies:
        c.wait()
    pltpu.sync_copy(rows_v, out_hbm.at[pl.ds(base, B_PER_W), :])


def _sc_gather(ids, tab):
    mesh = plsc.VectorSubcoreMesh(core_axis_name="c", subcore_axis_name="s",
                                  num_cores=NC, num_subcores=NS)
    f = pl.kernel(
        _gather_body,
        out_type=jax.ShapeDtypeStruct((BATCH, EMBED), jnp.float32),
        mesh=mesh,
        compiler_params=pltpu.CompilerParams(needs_layout_passes=False,
                                             use_tc_tiling_on_sc=False),
        scratch_types=[
            pltpu.VMEM((N_CHUNK, CHUNK), jnp.int32),
            pltpu.VMEM((B_PER_W, EMBED), jnp.float32),
            pltpu.SemaphoreType.DMA,
        ],
    )
    return f(ids, tab)


def _combine_body(u_ref, p_ref, wb_ref, out_ref):
    wrow = wb_ref[0, :EMBED]
    bias = wb_ref[0, EMBED]
    prod = u_ref[0] * p_ref[0] * wrow[None, :]
    out_ref[0, 0, :] = jnp.sum(prod, axis=1) + bias


def _tc_combine(u_rows, p_rows, wb):
    u3 = u_rows.reshape(N_BLK, ROWS_BLK, EMBED)
    p3 = p_rows.reshape(N_BLK, ROWS_BLK, EMBED)
    out = pl.pallas_call(
        _combine_body,
        grid=(N_BLK,),
        in_specs=[
            pl.BlockSpec((1, ROWS_BLK, EMBED), lambda i: (i, 0, 0)),
            pl.BlockSpec((1, ROWS_BLK, EMBED), lambda i: (i, 0, 0)),
            pl.BlockSpec((1, EMBED + 16), lambda i: (0, 0)),
        ],
        out_specs=pl.BlockSpec((1, 1, ROWS_BLK), lambda i: (i, 0, 0)),
        out_shape=jax.ShapeDtypeStruct((N_BLK, 1, ROWS_BLK), jnp.float32),
    )(u3, p3, wb)
    return out.reshape(BATCH)


@jax.jit
def _run(uid, pid, utab, ptab, wb):
    u_rows = _sc_gather(uid, utab)
    p_rows = _sc_gather(pid, ptab)
    return _tc_combine(u_rows, p_rows, wb)


def kernel(user_ids, product_ids, user_embedding, product_embedding, fc_w, fc_b):
    uid = user_ids.astype(jnp.int32)
    pid = product_ids.astype(jnp.int32)
    wb = jnp.concatenate(
        [fc_w.reshape(EMBED), jnp.broadcast_to(fc_b.reshape(1), (16,))])
    return _run(uid, pid, user_embedding, product_embedding,
                wb.reshape(1, EMBED + 16))


# speedup vs baseline: 1.3859x; 1.3676x over previous
import functools

import jax
import jax.numpy as jnp
from jax import lax
from jax.experimental import pallas as pl
from jax.experimental.pallas import tpu as pltpu
from jax.experimental.pallas import tpu_sc as plsc

BATCH = 16384
EMBED = 64
NC = 2
NS = 16
NW = NC * NS

OUT_ROWS = 17408
ROWS_BLK = 1024
N_BLK = BATCH // ROWS_BLK

I32 = jnp.int32


def _scatter_gather_call(n_rows, shift, g_slabs):
    gcols = 128 * g_slabs
    n_groups = n_rows // gcols + 1
    nb_base = n_groups // NW
    nb_rem = n_groups % NW
    slab_bytes = EMBED * 128 * 4
    row_bytes = 16 * 128 * 4

    def body(ids_hbm, tab_hbm, tail_hbm, out_hbm,
             idsv, hpos, spos, counts, offs, cursor, slabbuf, outbuf,
             sem, sem2):
        w = lax.axis_index("s") * NC + lax.axis_index("c")
        lo = w * nb_base + jnp.minimum(w, nb_rem)
        nb = nb_base + jnp.where(w < nb_rem, 1, 0)
        iota = lax.iota(I32, 16)
        ones = jnp.ones((16,), I32)
        zeros = jnp.zeros((16,), I32)

        pltpu.sync_copy(ids_hbm, idsv)
        for c in range(9):
            counts[pl.ds(c * 16, 16)] = zeros

        def p1(v, kh):
            u = idsv[pl.ds(v * 16, 16)]
            bg = u >> shift
            m = (bg >= lo) & (bg < lo + nb)
            plsc.store_compressed(hpos.at[pl.ds(kh, 16)], v * 16 + iota,
                                  mask=m)
            bl = jnp.where(m, bg - lo, 0)
            plsc.addupdate_scatter(counts, [bl], ones, mask=m)
            return kh + plsc.all_reduce_population_count(m)[0]

        kh = lax.fori_loop(0, BATCH // 16, p1, jnp.asarray(0, I32))

        car = jnp.asarray(0, I32)
        for c in range(9):
            v = counts[pl.ds(c * 16, 16)]
            s = plsc.cumsum(v)
            e = s - v + car
            offs[pl.ds(c * 16, 16)] = e
            cursor[pl.ds(c * 16, 16)] = e
            car = car + s[15]

        def p3(k, carry):
            base = k * 16
            valid = (base + iota) < kh
            p = hpos[pl.ds(base, 16)] & (BATCH - 1)
            u = plsc.load_gather(idsv, [p])
            bl = jnp.where(valid, (u >> shift) - lo, 0)
            boff = plsc.load_gather(cursor, [bl])
            dup, lastm = plsc.scan_count(bl, valid)
            plsc.store_scatter(spos, [boff + dup - 1], p, mask=valid)
            plsc.addupdate_scatter(cursor, [bl], dup,
                                   mask=lastm & valid)
            return carry

        lax.fori_loop(0, (kh + 15) >> 4, p3, jnp.asarray(0, I32))

        def fire(g_rel, parity):
            g = lo + g_rel
            is_tail = g == n_groups - 1

            @pl.when(jnp.logical_not(is_tail))
            def _():
                for j in range(g_slabs):
                    pltpu.async_copy(
                        tab_hbm.at[:, pl.ds((g * g_slabs + j) * 128, 128)],
                        slabbuf.at[parity, j], sem)

            @pl.when(is_tail)
            def _():
                for j in range(g_slabs):
                    pltpu.async_copy(tail_hbm, slabbuf.at[parity, j], sem)

        def group_step(g_rel, wcnt):
            parity = g_rel & 1

            @pl.when(g_rel + 1 < nb)
            def _():
                fire(g_rel + 1, (g_rel + 1) & 1)

            for j in range(g_slabs):
                pltpu.make_async_copy(
                    tab_hbm.at[:, pl.ds(0, 128)],
                    slabbuf.at[parity, j], sem).wait()

            ov = plsc.load_gather(offs, [jnp.minimum(g_rel + iota, 143)])
            st, en = ov[0], ov[1]
            pb = jnp.full((16,), parity, I32)

            def window(k, wc):
                base = st + k * 16
                valid = (base + iota) < en
                hp = spos[pl.ds(base, 16)] & (BATCH - 1)
                hu = plsc.load_gather(idsv, [hp])
                lane = hu & 127
                sg = (hu >> 7) & (g_slabs - 1)

                rowv = (wc & 3) * 16 + iota
                for d in range(EMBED):
                    dsp = jnp.full((16,), d, I32)
                    val = plsc.load_gather(slabbuf, [pb, sg, dsp, lane])
                    plsc.store_scatter(outbuf, [rowv, dsp], val)
                rows_dst = jnp.where(valid, hp, 16384 + iota)
                pltpu.async_copy(outbuf.at[pl.ds((wc & 3) * 16, 16), :],
                                 out_hbm.at[rows_dst], sem2)
                pltpu.make_async_copy(outbuf.at[pl.ds((wc & 3) * 16, 16), :],
                                      out_hbm.at[rows_dst], sem2).wait()
                return wc + 1

            nwin = (en - st + 15) >> 4
            return lax.fori_loop(0, nwin, window, wcnt)

        fire(0, 0)
        lax.fori_loop(0, nb, group_step, jnp.asarray(0, I32))

    mesh = plsc.VectorSubcoreMesh(core_axis_name="c", subcore_axis_name="s",
                                  num_cores=NC, num_subcores=NS)
    return pl.kernel(
        body,
        out_type=jax.ShapeDtypeStruct((OUT_ROWS, 128), jnp.float32),
        mesh=mesh,
        compiler_params=pltpu.CompilerParams(needs_layout_passes=False,
                                             use_tc_tiling_on_sc=True),
        scratch_types=[
            pltpu.VMEM((BATCH,), I32),
            pltpu.VMEM((BATCH + 16,), I32),
            pltpu.VMEM((BATCH + 16,), I32),
            pltpu.VMEM((144,), I32),
            pltpu.VMEM((144,), I32),
            pltpu.VMEM((144,), I32),
            pltpu.VMEM((2, g_slabs, EMBED, 128), jnp.float32),
            pltpu.VMEM((64, 128), jnp.float32),
            pltpu.SemaphoreType.DMA,
            pltpu.SemaphoreType.DMA,
        ],
    )


def _combine_body(u_ref, p_ref, wb_ref, out_ref):
    wrow = wb_ref[0, :EMBED]
    bias = wb_ref[0, EMBED]
    prod = u_ref[0][:, :EMBED] * p_ref[0][:, :EMBED] * wrow[None, :]
    out_ref[0, 0, :] = jnp.sum(prod, axis=1) + bias


def _tc_combine(u_rows, p_rows, wb):
    u3 = u_rows.reshape(OUT_ROWS // ROWS_BLK, ROWS_BLK, 128)
    p3 = p_rows.reshape(OUT_ROWS // ROWS_BLK, ROWS_BLK, 128)
    out = pl.pallas_call(
        _combine_body,
        grid=(N_BLK,),
        in_specs=[
            pl.BlockSpec((1, ROWS_BLK, 128), lambda i: (i, 0, 0)),
            pl.BlockSpec((1, ROWS_BLK, 128), lambda i: (i, 0, 0)),
            pl.BlockSpec((1, EMBED + 16), lambda i: (0, 0)),
        ],
        out_specs=pl.BlockSpec((1, 1, ROWS_BLK), lambda i: (i, 0, 0)),
        out_shape=jax.ShapeDtypeStruct((N_BLK, 1, ROWS_BLK), jnp.float32),
    )(u3, p3, wb)
    return out.reshape(BATCH)


@jax.jit
def _run(uid, pid, utab_t, ptab_t, utail, ptail, wb):
    u_rows = _scatter_gather_call(1000000, 8, 2)(uid, utab_t, utail)
    p_rows = _scatter_gather_call(100000, 7, 1)(pid, ptab_t, ptail)
    return _tc_combine(u_rows, p_rows, wb)


def kernel(user_ids, product_ids, user_embedding, product_embedding, fc_w, fc_b):
    uid = user_ids.astype(I32)
    pid = product_ids.astype(I32)
    n_u, n_p = user_embedding.shape[0], product_embedding.shape[0]
    utail = jnp.pad(user_embedding[n_u - n_u % 256:].T,
                    ((0, 0), (0, 128 - n_u % 256)))
    ptail = jnp.pad(product_embedding[n_p - n_p % 128:].T,
                    ((0, 0), (0, 128 - n_p % 128)))
    wb = jnp.concatenate(
        [fc_w.reshape(EMBED), jnp.broadcast_to(fc_b.reshape(1), (16,))])
    return _run(uid, pid, user_embedding.T, product_embedding.T,
                utail, ptail, wb.reshape(1, EMBED + 16))


# speedup vs baseline: 1.9621x; 1.4157x over previous
import functools

import jax
import jax.numpy as jnp
from jax import lax
from jax.experimental import pallas as pl
from jax.experimental.pallas import tpu as pltpu
from jax.experimental.pallas import tpu_sc as plsc

BATCH = 16384
EMBED = 64
NC = 2
NS = 16
NW = NC * NS

GCOLS = 512
SHIFT = 9
OUT_ROWS = 17408
ROWS_BLK = 1024
N_BLK = BATCH // ROWS_BLK
STRIP = BATCH // 16

I32 = jnp.int32


def _gather_call(n_rows):
    n_groups = n_rows // GCOLS + 1
    nb_base = n_groups // NW
    nb_rem = n_groups % NW

    def body(ids_hbm, tab_hbm, tail_hbm, out_hbm,
             idsv, hpos, spos, counts, offs, cursor, slabbuf, outbuf,
             sem, sem2):
        w = lax.axis_index("s") * NC + lax.axis_index("c")
        lo = w * nb_base + jnp.minimum(w, nb_rem)
        nb = nb_base + jnp.where(w < nb_rem, 1, 0)
        iota = lax.iota(I32, 16)
        ones = jnp.ones((16,), I32)
        zeros = jnp.zeros((16,), I32)
        strip_base = iota * STRIP

        def fire(g_rel, parity):
            g = lo + g_rel
            is_tail = g == n_groups - 1

            @pl.when(jnp.logical_not(is_tail))
            def _():
                pltpu.async_copy(tab_hbm.at[:, pl.ds(g * GCOLS, GCOLS)],
                                 slabbuf.at[parity], sem)

            @pl.when(is_tail)
            def _():
                pltpu.async_copy(tail_hbm, slabbuf.at[parity], sem)

        fire(0, 0)

        @pl.when(nb > 1)
        def _():
            fire(1, 1)

        pltpu.sync_copy(ids_hbm, idsv)
        for c in range(9):
            counts[pl.ds(c * 16, 16)] = zeros

        def p1(k, cur):
            u = plsc.load_gather(idsv, [strip_base + k])
            bg = u >> SHIFT
            m = (bg >= lo) & (bg < lo + nb)
            bl = jnp.where(m, bg - lo, 0)
            plsc.addupdate_scatter(counts, [bl], ones, mask=m)
            plsc.store_scatter(hpos, [strip_base + cur], strip_base + k,
                               mask=m)
            return cur + jnp.where(m, 1, 0)

        cur16 = lax.fori_loop(0, STRIP, p1, jnp.zeros((16,), I32))

        car = jnp.asarray(0, I32)
        for c in range(9):
            v = counts[pl.ds(c * 16, 16)]
            s = plsc.cumsum(v)
            e = s - v + car
            offs[pl.ds(c * 16, 16)] = e
            cursor[pl.ds(c * 16, 16)] = e
            car = car + s[15]

        def strip(j, carry):
            cnt = cur16[j]

            def p3(k, carry2):
                base = j * STRIP + k * 16
                valid = (k * 16 + iota) < cnt
                p = hpos[pl.ds(base, 16)] & (BATCH - 1)
                u = plsc.load_gather(idsv, [p])
                bl = jnp.where(valid, (u >> SHIFT) - lo, 0)
                boff = plsc.load_gather(cursor, [bl])
                dup, lastm = plsc.scan_count(bl, valid)
                plsc.store_scatter(spos, [boff + dup - 1], p, mask=valid)
                plsc.addupdate_scatter(cursor, [bl], dup,
                                       mask=lastm & valid)
                return carry2

            lax.fori_loop(0, (cnt + 15) >> 4, p3, jnp.asarray(0, I32))
            return carry

        for j in range(16):
            strip(j, 0)

        def group_step(g_rel, wcnt):
            parity = g_rel & 1
            pltpu.make_async_copy(tail_hbm, slabbuf.at[parity], sem).wait()

            ov = plsc.load_gather(offs, [jnp.minimum(g_rel + iota, 143)])
            st, en = ov[0], ov[1]
            pb = jnp.full((16,), parity, I32)

            def window(k, wc):
                base = st + k * 16
                valid = (base + iota) < en
                hp = spos[pl.ds(base, 16)] & (BATCH - 1)
                hu = plsc.load_gather(idsv, [hp])
                lane = hu & (GCOLS - 1)

                @pl.when(wc >= 4)
                def _():
                    pltpu.make_async_copy(
                        outbuf.at[pl.ds(0, 16), :],
                        out_hbm.at[16384 + iota], sem2).wait()

                rowv = (wc & 3) * 16 + iota
                for d in range(EMBED):
                    dsp = jnp.full((16,), d, I32)
                    val = plsc.load_gather(slabbuf, [pb, dsp, lane])
                    plsc.store_scatter(outbuf, [rowv, dsp], val)
                rows_dst = jnp.where(valid, hp, 16384 + iota)
                pltpu.async_copy(outbuf.at[pl.ds((wc & 3) * 16, 16), :],
                                 out_hbm.at[rows_dst], sem2)
                return wc + 1

            nwin = (en - st + 15) >> 4
            wcnt = lax.fori_loop(0, nwin, window, wcnt)

            @pl.when(g_rel + 2 < nb)
            def _():
                fire(g_rel + 2, parity)

            return wcnt

        wcnt = lax.fori_loop(0, nb, group_step, jnp.asarray(0, I32))

        def drain(_, c):
            pltpu.make_async_copy(outbuf.at[pl.ds(0, 16), :],
                                  out_hbm.at[16384 + iota], sem2).wait()
            return c

        lax.fori_loop(0, jnp.minimum(wcnt, 4), drain, jnp.asarray(0, I32))

    mesh = plsc.VectorSubcoreMesh(core_axis_name="c", subcore_axis_name="s",
                                  num_cores=NC, num_subcores=NS)
    return pl.kernel(
        body,
        out_type=jax.ShapeDtypeStruct((OUT_ROWS, 128), jnp.float32),
        mesh=mesh,
        compiler_params=pltpu.CompilerParams(needs_layout_passes=False,
                                             use_tc_tiling_on_sc=True),
        scratch_types=[
            pltpu.VMEM((BATCH,), I32),
            pltpu.VMEM((BATCH,), I32),
            pltpu.VMEM((BATCH + 16,), I32),
            pltpu.VMEM((144,), I32),
            pltpu.VMEM((144,), I32),
            pltpu.VMEM((144,), I32),
            pltpu.VMEM((2, EMBED, GCOLS), jnp.float32),
            pltpu.VMEM((64, 128), jnp.float32),
            pltpu.SemaphoreType.DMA,
            pltpu.SemaphoreType.DMA,
        ],
    )


def _combine_body(u_ref, p_ref, wb_ref, out_ref):
    wrow = wb_ref[0, :EMBED]
    bias = wb_ref[0, EMBED]
    prod = u_ref[0][:, :EMBED] * p_ref[0][:, :EMBED] * wrow[None, :]
    out_ref[0, 0, :] = jnp.sum(prod, axis=1) + bias


def _tc_combine(u_rows, p_rows, wb):
    u3 = u_rows.reshape(OUT_ROWS // ROWS_BLK, ROWS_BLK, 128)
    p3 = p_rows.reshape(OUT_ROWS // ROWS_BLK, ROWS_BLK, 128)
    out = pl.pallas_call(
        _combine_body,
        grid=(N_BLK,),
        in_specs=[
            pl.BlockSpec((1, ROWS_BLK, 128), lambda i: (i, 0, 0)),
            pl.BlockSpec((1, ROWS_BLK, 128), lambda i: (i, 0, 0)),
            pl.BlockSpec((1, EMBED + 16), lambda i: (0, 0)),
        ],
        out_specs=pl.BlockSpec((1, 1, ROWS_BLK), lambda i: (i, 0, 0)),
        out_shape=jax.ShapeDtypeStruct((N_BLK, 1, ROWS_BLK), jnp.float32),
    )(u3, p3, wb)
    return out.reshape(BATCH)


@jax.jit
def _run(uid, pid, utab_t, ptab_t, utail, ptail, wb):
    u_rows = _gather_call(1000000)(uid, utab_t, utail)
    p_rows = _gather_call(100000)(pid, ptab_t, ptail)
    return _tc_combine(u_rows, p_rows, wb)


def kernel(user_ids, product_ids, user_embedding, product_embedding, fc_w, fc_b):
    uid = user_ids.astype(I32)
    pid = product_ids.astype(I32)
    n_u, n_p = user_embedding.shape[0], product_embedding.shape[0]
    utail = jnp.pad(user_embedding[n_u - n_u % GCOLS:].T,
                    ((0, 0), (0, GCOLS - n_u % GCOLS)))
    ptail = jnp.pad(product_embedding[n_p - n_p % GCOLS:].T,
                    ((0, 0), (0, GCOLS - n_p % GCOLS)))
    wb = jnp.concatenate(
        [fc_w.reshape(EMBED), jnp.broadcast_to(fc_b.reshape(1), (16,))])
    return _run(uid, pid, user_embedding.T, product_embedding.T,
                utail, ptail, wb.reshape(1, EMBED + 16))
